# baseline (speedup 1.0000x reference)
import jax
import jax.numpy as jnp
from jax import lax
from jax.experimental import pallas as pl
from jax.experimental.pallas import tpu as pltpu
from jax.experimental.pallas import tpu_sc as plsc

B, N, V, F_ = 2, 2048, 6890, 13776
Fp = 13824
VF = V * 3
VFP = 20672
NC, NS = 2, 16
NW = NC * NS
FPW = Fp // NW
PPW = (B * N) // NW
ROWS = 24

PB = 8
FB = 128
UNROLL = 2
NSTEP = Fp // (FB * UNROLL)
BIGI = 2 ** 30

_SC_PARAMS = pltpu.CompilerParams(
    needs_layout_passes=False, use_tc_tiling_on_sc=False
)


def _sc_gather_body(vflat_hbm, fidx_hbm, tri_hbm, table_v, fidx_v, out_v):
    wid = lax.axis_index("s") * NC + lax.axis_index("c")
    lane = lax.iota(jnp.int32, 16)
    fbase = wid * FPW
    pltpu.sync_copy(fidx_hbm.at[pl.ds(fbase * 3, FPW * 3)], fidx_v)
    for b in range(B):
        pltpu.sync_copy(vflat_hbm.at[pl.ds(b * VFP, VFP)], table_v)
        for i in range(FPW // 16):
            va = plsc.load_gather(fidx_v, [lane * 3 + (i * 48 + 0)])
            vb = plsc.load_gather(fidx_v, [lane * 3 + (i * 48 + 1)])
            vc = plsc.load_gather(fidx_v, [lane * 3 + (i * 48 + 2)])
            ax = plsc.load_gather(table_v, [va * 3])
            ay = plsc.load_gather(table_v, [va * 3 + 1])
            az = plsc.load_gather(table_v, [va * 3 + 2])
            bx = plsc.load_gather(table_v, [vb * 3])
            by = plsc.load_gather(table_v, [vb * 3 + 1])
            bz = plsc.load_gather(table_v, [vb * 3 + 2])
            cx = plsc.load_gather(table_v, [vc * 3])
            cy = plsc.load_gather(table_v, [vc * 3 + 1])
            cz = plsc.load_gather(table_v, [vc * 3 + 2])
            abx = bx - ax; aby = by - ay; abz = bz - az
            acx = cx - ax; acy = cy - ay; acz = cz - az
            fnx = aby * acz - abz * acy
            fny = abz * acx - abx * acz
            fnz = abx * acy - aby * acx
            vals = (ax, ay, az, bx, by, bz, cx, cy, cz,
                    fnx, fny, fnz, abx, aby, abz, acx, acy, acz)
            for r, val in enumerate(vals):
                out_v[r, pl.ds(i * 16, 16)] = val
        for r in range(18):
            pltpu.sync_copy(
                out_v.at[r],
                tri_hbm.at[pl.ds((b * ROWS + r) * Fp + fbase, FPW)],
            )


def _sc_gather(vflat, fidx_flat):
    kfn = pl.kernel(
        _sc_gather_body,
        out_type=jax.ShapeDtypeStruct((B * ROWS * Fp,), jnp.float32),
        mesh=plsc.VectorSubcoreMesh(core_axis_name="c", subcore_axis_name="s"),
        compiler_params=_SC_PARAMS,
        scratch_types=[
            pltpu.VMEM((VFP,), jnp.float32),
            pltpu.VMEM((FPW * 3,), jnp.int32),
            pltpu.VMEM((ROWS, FPW), jnp.float32),
        ],
    )
    return kfn(vflat, fidx_flat).reshape(B, ROWS, Fp)


def _safe(x):
    return jnp.where(jnp.abs(x) < 1e-12, 1e-12, x)


def _tc_body(coords_ref, tri_ref, misc_ref):
    c = coords_ref[0]
    px = jnp.broadcast_to(c[:, 0:1], (PB, FB))
    py = jnp.broadcast_to(c[:, 1:2], (PB, FB))
    pz = jnp.broadcast_to(c[:, 2:3], (PB, FB))

    def one_block(kb, st):
        bd2, bu, bv, bw, bhx, bhy, bhz, bsg, bk = st
        row = lambda r: tri_ref[0, r, :, pl.ds(kb * FB, FB)]
        ax = row(0); ay = row(1); az = row(2)
        bx = row(3); by = row(4); bz = row(5)
        cx = row(6); cy = row(7); cz = row(8)
        fnx = row(9); fny = row(10); fnz = row(11)
        abx = row(12); aby = row(13); abz = row(14)
        acx = row(15); acy = row(16); acz = row(17)

        apx = px - ax; apy = py - ay; apz = pz - az
        d1 = (abx * apx + aby * apy) + abz * apz
        d2_ = (acx * apx + acy * apy) + acz * apz
        bpx = px - bx; bpy = py - by; bpz = pz - bz
        d3 = (abx * bpx + aby * bpy) + abz * bpz
        d4 = (acx * bpx + acy * bpy) + acz * bpz
        cpx_ = px - cx; cpy_ = py - cy; cpz_ = pz - cz
        d5 = (abx * cpx_ + aby * cpy_) + abz * cpz_
        d6 = (acx * cpx_ + acy * cpy_) + acz * cpz_

        vc = d1 * d4 - d3 * d2_
        vb = d5 * d2_ - d1 * d6
        va = d3 * d6 - d5 * d4
        v_ab = d1 / _safe(d1 - d3)
        w_ac = d2_ / _safe(d2_ - d6)
        w_bc = (d4 - d3) / _safe((d4 - d3) + (d5 - d6))
        denom = _safe((va + vb) + vc)
        v_in = vb / denom
        w_in = vc / denom
        u = (1.0 - v_in) - w_in
        v = v_in
        w = w_in
        c6 = (va <= 0) & ((d4 - d3) >= 0) & ((d5 - d6) >= 0)
        u = jnp.where(c6, 0.0, u); v = jnp.where(c6, 1.0 - w_bc, v); w = jnp.where(c6, w_bc, w)
        c5 = (vb <= 0) & (d2_ >= 0) & (d6 <= 0)
        u = jnp.where(c5, 1.0 - w_ac, u); v = jnp.where(c5, 0.0, v); w = jnp.where(c5, w_ac, w)
        c4 = (d6 >= 0) & (d5 <= d6)
        u = jnp.where(c4, 0.0, u); v = jnp.where(c4, 0.0, v); w = jnp.where(c4, 1.0, w)
        c3 = (vc <= 0) & (d1 >= 0) & (d3 <= 0)
        u = jnp.where(c3, 1.0 - v_ab, u); v = jnp.where(c3, v_ab, v); w = jnp.where(c3, 0.0, w)
        c2 = (d3 >= 0) & (d4 <= d3)
        u = jnp.where(c2, 0.0, u); v = jnp.where(c2, 1.0, v); w = jnp.where(c2, 0.0, w)
        c1 = (d1 <= 0) & (d2_ <= 0)
        u = jnp.where(c1, 1.0, u); v = jnp.where(c1, 0.0, v); w = jnp.where(c1, 0.0, w)

        hx = (u * ax + v * bx) + w * cx
        hy = (u * ay + v * by) + w * cy
        hz = (u * az + v * bz) + w * cz
        dx = px - hx; dy = py - hy; dz = pz - hz
        d2 = (dx * dx + dy * dy) + dz * dz
        sgnd = (dx * fnx + dy * fny) + dz * fnz
        sgn = jnp.where(sgnd < 0.0, -1.0, 1.0)

        upd = d2 < bd2
        return (
            jnp.where(upd, d2, bd2),
            jnp.where(upd, u, bu),
            jnp.where(upd, v, bv),
            jnp.where(upd, w, bw),
            jnp.where(upd, hx, bhx),
            jnp.where(upd, hy, bhy),
            jnp.where(upd, hz, bhz),
            jnp.where(upd, sgn, bsg),
            jnp.where(upd, kb, bk),
        )

    def step(k, st):
        for uu in range(UNROLL):
            st = one_block(k * UNROLL + uu, st)
        return st

    finit = lambda fill: jnp.full((PB, FB), fill, jnp.float32)
    init = (finit(jnp.inf), finit(0), finit(0), finit(0), finit(0), finit(0),
            finit(0), finit(0), jnp.zeros((PB, FB), jnp.int32))
    bd2, bu, bv, bw, bhx, bhy, bhz, bsg, bk = lax.fori_loop(0, NSTEP, step, init)

    lane = lax.broadcasted_iota(jnp.int32, (PB, FB), 1)
    bfid = bk * FB + lane
    m = jnp.min(bd2, axis=1, keepdims=True)
    cand = jnp.where(bd2 == m, bfid, BIGI)
    wfid = jnp.min(cand, axis=1, keepdims=True)
    onehot = bfid == wfid
    sel = lambda val: jnp.sum(jnp.where(onehot, val, 0.0), axis=1, keepdims=True)
    su = sel(bu); sv = sel(bv); sw = sel(bw)
    shx = sel(bhx); shy = sel(bhy); shz = sel(bhz)
    ssg = sel(bsg)

    p0x = c[:, 0:1]; p0y = c[:, 1:2]; p0z = c[:, 2:3]
    dist = jnp.sqrt(m + 1e-12)
    sdf = dist * ssg
    dx = shx - p0x; dy = shy - p0y; dz = shz - p0z
    nrm = jnp.sqrt((dx * dx + dy * dy) + dz * dz)
    den = jnp.maximum(nrm, 1e-6)
    misc_ref[0] = jnp.concatenate(
        [sdf, su, sv, sw, wfid.astype(jnp.float32), dx / den, dy / den, dz / den],
        axis=1,
    )


def _tc_query(coords_pad, tri_pack, interpret=False):
    return pl.pallas_call(
        _tc_body,
        grid=(B, N // PB),
        in_specs=[
            pl.BlockSpec((1, PB, 8), lambda b, n: (b, n, 0)),
            pl.BlockSpec((1, ROWS, 8, Fp), lambda b, n: (b, 0, 0, 0)),
        ],
        out_specs=pl.BlockSpec((1, PB, 8), lambda b, n: (b, n, 0)),
        out_shape=jax.ShapeDtypeStruct((B, N, 8), jnp.float32),
        interpret=interpret,
    )(coords_pad, tri_pack)


def _sc_combine_body(canv_hbm, fidx_hbm, fid_hbm, wu_hbm, wv_hbm, ww_hbm,
                     ox_hbm, oy_hbm, oz_hbm,
                     canv_v, fidx_v, fid_v, wu_v, wv_v, ww_v, ox_v, oy_v, oz_v):
    wid = lax.axis_index("s") * NC + lax.axis_index("c")
    pltpu.sync_copy(canv_hbm, canv_v)
    pltpu.sync_copy(fidx_hbm, fidx_v)
    base = wid * PPW
    pltpu.sync_copy(fid_hbm.at[pl.ds(base, PPW)], fid_v)
    pltpu.sync_copy(wu_hbm.at[pl.ds(base, PPW)], wu_v)
    pltpu.sync_copy(wv_hbm.at[pl.ds(base, PPW)], wv_v)
    pltpu.sync_copy(ww_hbm.at[pl.ds(base, PPW)], ww_v)
    outs = (ox_v, oy_v, oz_v)
    for i in range(PPW // 16):
        f = fid_v[pl.ds(i * 16, 16)]
        u = wu_v[pl.ds(i * 16, 16)]
        v = wv_v[pl.ds(i * 16, 16)]
        w = ww_v[pl.ds(i * 16, 16)]
        ia = plsc.load_gather(fidx_v, [f * 3])
        ib = plsc.load_gather(fidx_v, [f * 3 + 1])
        ic = plsc.load_gather(fidx_v, [f * 3 + 2])
        for cdim in range(3):
            ca = plsc.load_gather(canv_v, [ia * 3 + cdim])
            cb = plsc.load_gather(canv_v, [ib * 3 + cdim])
            cc = plsc.load_gather(canv_v, [ic * 3 + cdim])
            outs[cdim][pl.ds(i * 16, 16)] = (u * ca + v * cb) + w * cc
    pltpu.sync_copy(ox_v, ox_hbm.at[pl.ds(base, PPW)])
    pltpu.sync_copy(oy_v, oy_hbm.at[pl.ds(base, PPW)])
    pltpu.sync_copy(oz_v, oz_hbm.at[pl.ds(base, PPW)])


def _sc_combine(canv_flat, fidx_flat, fid_flat, wu, wv, ww):
    kfn = pl.kernel(
        _sc_combine_body,
        out_type=[jax.ShapeDtypeStruct((B * N,), jnp.float32)] * 3,
        mesh=plsc.VectorSubcoreMesh(core_axis_name="c", subcore_axis_name="s"),
        compiler_params=_SC_PARAMS,
        scratch_types=[
            pltpu.VMEM((VFP,), jnp.float32),
            pltpu.VMEM((Fp * 3,), jnp.int32),
            pltpu.VMEM((PPW,), jnp.int32),
            pltpu.VMEM((PPW,), jnp.float32),
            pltpu.VMEM((PPW,), jnp.float32),
            pltpu.VMEM((PPW,), jnp.float32),
            pltpu.VMEM((PPW,), jnp.float32),
            pltpu.VMEM((PPW,), jnp.float32),
            pltpu.VMEM((PPW,), jnp.float32),
        ],
    )
    ox, oy, oz = kfn(canv_flat, fidx_flat, fid_flat, wu, wv, ww)
    return jnp.stack([ox, oy, oz], axis=-1)


@jax.jit
def kernel(coords, smpl_V, can_V, smpl_F):
    fidx_pad = jnp.concatenate(
        [smpl_F, jnp.broadcast_to(smpl_F[0:1], (Fp - F_, 3))], axis=0
    ).reshape(-1)
    vflat = jnp.pad(smpl_V.reshape(B, VF), ((0, 0), (0, VFP - VF))).reshape(-1)
    canv_flat = jnp.pad(can_V.reshape(VF), (0, VFP - VF))
    coords_pad = jnp.pad(coords, ((0, 0), (0, 0), (0, 5)))

    tri_pack = _sc_gather(vflat, fidx_pad)
    tri4 = jnp.broadcast_to(tri_pack[:, :, None, :], (B, ROWS, 8, Fp))
    misc = _tc_query(coords_pad, tri4)

    sdf = misc[:, :, 0]
    wu = misc[:, :, 1].reshape(-1)
    wv = misc[:, :, 2].reshape(-1)
    ww = misc[:, :, 3].reshape(-1)
    fid_flat = misc[:, :, 4].astype(jnp.int32).reshape(-1)
    normal = misc[:, :, 5:8]

    out_coord = _sc_combine(canv_flat, fidx_pad, fid_flat, wu, wv, ww)
    out_coord = out_coord.reshape(B, N, 3)
    z = coords[..., 2:3]
    return (out_coord, sdf, normal, z)


# speedup vs baseline: 1.1101x; 1.1101x over previous
import jax
import jax.numpy as jnp
from jax import lax
from jax.experimental import pallas as pl
from jax.experimental.pallas import tpu as pltpu
from jax.experimental.pallas import tpu_sc as plsc

B, N, V, F_ = 2, 2048, 6890, 13776
Fp = 13824
VF = V * 3
VFP = 20672
NC, NS = 2, 16
NW = NC * NS
FPW = Fp // NW
PPW = (B * N) // NW
ROWS = 24

PB = 8
FB = 128
UNROLL = 4
NSTEP = Fp // (FB * UNROLL)
BIGI = 2 ** 30

_SC_PARAMS = pltpu.CompilerParams(
    needs_layout_passes=False, use_tc_tiling_on_sc=False
)


def _sc_gather_body(vflat_hbm, fidx_hbm, tri_hbm, table_v, fidx_v, out_v):
    wid = lax.axis_index("s") * NC + lax.axis_index("c")
    lane = lax.iota(jnp.int32, 16)
    fbase = wid * FPW
    pltpu.sync_copy(fidx_hbm.at[pl.ds(fbase * 3, FPW * 3)], fidx_v)
    for b in range(B):
        pltpu.sync_copy(vflat_hbm.at[pl.ds(b * VFP, VFP)], table_v)
        for i in range(FPW // 16):
            va = plsc.load_gather(fidx_v, [lane * 3 + (i * 48 + 0)])
            vb = plsc.load_gather(fidx_v, [lane * 3 + (i * 48 + 1)])
            vc = plsc.load_gather(fidx_v, [lane * 3 + (i * 48 + 2)])
            ax = plsc.load_gather(table_v, [va * 3])
            ay = plsc.load_gather(table_v, [va * 3 + 1])
            az = plsc.load_gather(table_v, [va * 3 + 2])
            bx = plsc.load_gather(table_v, [vb * 3])
            by = plsc.load_gather(table_v, [vb * 3 + 1])
            bz = plsc.load_gather(table_v, [vb * 3 + 2])
            cx = plsc.load_gather(table_v, [vc * 3])
            cy = plsc.load_gather(table_v, [vc * 3 + 1])
            cz = plsc.load_gather(table_v, [vc * 3 + 2])
            abx = bx - ax; aby = by - ay; abz = bz - az
            acx = cx - ax; acy = cy - ay; acz = cz - az
            fnx = aby * acz - abz * acy
            fny = abz * acx - abx * acz
            fnz = abx * acy - aby * acx
            vals = (ax, ay, az, bx, by, bz, cx, cy, cz,
                    fnx, fny, fnz, abx, aby, abz, acx, acy, acz)
            for r, val in enumerate(vals):
                out_v[r, pl.ds(i * 16, 16)] = val
        for r in range(18):
            pltpu.sync_copy(
                out_v.at[r],
                tri_hbm.at[pl.ds((b * ROWS + r) * Fp + fbase, FPW)],
            )


def _sc_gather(vflat, fidx_flat):
    kfn = pl.kernel(
        _sc_gather_body,
        out_type=jax.ShapeDtypeStruct((B * ROWS * Fp,), jnp.float32),
        mesh=plsc.VectorSubcoreMesh(core_axis_name="c", subcore_axis_name="s"),
        compiler_params=_SC_PARAMS,
        scratch_types=[
            pltpu.VMEM((VFP,), jnp.float32),
            pltpu.VMEM((FPW * 3,), jnp.int32),
            pltpu.VMEM((ROWS, FPW), jnp.float32),
        ],
    )
    return kfn(vflat, fidx_flat).reshape(B, ROWS, Fp)


def _safe(x):
    return jnp.where(jnp.abs(x) < 1e-12, 1e-12, x)


def _tc_body(coords_ref, tri_ref, misc_ref):
    c = coords_ref[0]
    px = jnp.broadcast_to(c[:, 0:1], (PB, FB))
    py = jnp.broadcast_to(c[:, 1:2], (PB, FB))
    pz = jnp.broadcast_to(c[:, 2:3], (PB, FB))

    def one_block(kb, st):
        bd2, bu, bv, bw, bhx, bhy, bhz, bsg, bk = st
        ts = tri_ref[0, :, pl.ds(kb * FB, FB)]
        row = lambda r: ts[r:r + 1, :]
        ax = row(0); ay = row(1); az = row(2)
        bx = row(3); by = row(4); bz = row(5)
        cx = row(6); cy = row(7); cz = row(8)
        fnx = row(9); fny = row(10); fnz = row(11)
        abx = row(12); aby = row(13); abz = row(14)
        acx = row(15); acy = row(16); acz = row(17)

        apx = px - ax; apy = py - ay; apz = pz - az
        d1 = (abx * apx + aby * apy) + abz * apz
        d2_ = (acx * apx + acy * apy) + acz * apz
        bpx = px - bx; bpy = py - by; bpz = pz - bz
        d3 = (abx * bpx + aby * bpy) + abz * bpz
        d4 = (acx * bpx + acy * bpy) + acz * bpz
        cpx_ = px - cx; cpy_ = py - cy; cpz_ = pz - cz
        d5 = (abx * cpx_ + aby * cpy_) + abz * cpz_
        d6 = (acx * cpx_ + acy * cpy_) + acz * cpz_

        vc = d1 * d4 - d3 * d2_
        vb = d5 * d2_ - d1 * d6
        va = d3 * d6 - d5 * d4
        v_ab = d1 / _safe(d1 - d3)
        w_ac = d2_ / _safe(d2_ - d6)
        w_bc = (d4 - d3) / _safe((d4 - d3) + (d5 - d6))
        denom = _safe((va + vb) + vc)
        v_in = vb / denom
        w_in = vc / denom
        u = (1.0 - v_in) - w_in
        v = v_in
        w = w_in
        c6 = (va <= 0) & ((d4 - d3) >= 0) & ((d5 - d6) >= 0)
        u = jnp.where(c6, 0.0, u); v = jnp.where(c6, 1.0 - w_bc, v); w = jnp.where(c6, w_bc, w)
        c5 = (vb <= 0) & (d2_ >= 0) & (d6 <= 0)
        u = jnp.where(c5, 1.0 - w_ac, u); v = jnp.where(c5, 0.0, v); w = jnp.where(c5, w_ac, w)
        c4 = (d6 >= 0) & (d5 <= d6)
        u = jnp.where(c4, 0.0, u); v = jnp.where(c4, 0.0, v); w = jnp.where(c4, 1.0, w)
        c3 = (vc <= 0) & (d1 >= 0) & (d3 <= 0)
        u = jnp.where(c3, 1.0 - v_ab, u); v = jnp.where(c3, v_ab, v); w = jnp.where(c3, 0.0, w)
        c2 = (d3 >= 0) & (d4 <= d3)
        u = jnp.where(c2, 0.0, u); v = jnp.where(c2, 1.0, v); w = jnp.where(c2, 0.0, w)
        c1 = (d1 <= 0) & (d2_ <= 0)
        u = jnp.where(c1, 1.0, u); v = jnp.where(c1, 0.0, v); w = jnp.where(c1, 0.0, w)

        hx = (u * ax + v * bx) + w * cx
        hy = (u * ay + v * by) + w * cy
        hz = (u * az + v * bz) + w * cz
        dx = px - hx; dy = py - hy; dz = pz - hz
        d2 = (dx * dx + dy * dy) + dz * dz
        sgnd = (dx * fnx + dy * fny) + dz * fnz
        sgn = jnp.where(sgnd < 0.0, -1.0, 1.0)

        upd = d2 < bd2
        return (
            jnp.where(upd, d2, bd2),
            jnp.where(upd, u, bu),
            jnp.where(upd, v, bv),
            jnp.where(upd, w, bw),
            jnp.where(upd, hx, bhx),
            jnp.where(upd, hy, bhy),
            jnp.where(upd, hz, bhz),
            jnp.where(upd, sgn, bsg),
            jnp.where(upd, kb, bk),
        )

    def step(k, st):
        for uu in range(UNROLL):
            st = one_block(k * UNROLL + uu, st)
        return st

    finit = lambda fill: jnp.full((PB, FB), fill, jnp.float32)
    init = (finit(jnp.inf), finit(0), finit(0), finit(0), finit(0), finit(0),
            finit(0), finit(0), jnp.zeros((PB, FB), jnp.int32))
    bd2, bu, bv, bw, bhx, bhy, bhz, bsg, bk = lax.fori_loop(0, NSTEP, step, init)

    lane = lax.broadcasted_iota(jnp.int32, (PB, FB), 1)
    bfid = bk * FB + lane
    m = jnp.min(bd2, axis=1, keepdims=True)
    cand = jnp.where(bd2 == m, bfid, BIGI)
    wfid = jnp.min(cand, axis=1, keepdims=True)
    onehot = bfid == wfid
    sel = lambda val: jnp.sum(jnp.where(onehot, val, 0.0), axis=1, keepdims=True)
    su = sel(bu); sv = sel(bv); sw = sel(bw)
    shx = sel(bhx); shy = sel(bhy); shz = sel(bhz)
    ssg = sel(bsg)

    p0x = c[:, 0:1]; p0y = c[:, 1:2]; p0z = c[:, 2:3]
    dist = jnp.sqrt(m + 1e-12)
    sdf = dist * ssg
    dx = shx - p0x; dy = shy - p0y; dz = shz - p0z
    nrm = jnp.sqrt((dx * dx + dy * dy) + dz * dz)
    den = jnp.maximum(nrm, 1e-6)
    misc_ref[0] = jnp.concatenate(
        [sdf, su, sv, sw, wfid.astype(jnp.float32), dx / den, dy / den, dz / den],
        axis=1,
    )


def _tc_query(coords_pad, tri_pack, interpret=False):
    return pl.pallas_call(
        _tc_body,
        grid=(B, N // PB),
        in_specs=[
            pl.BlockSpec((1, PB, 8), lambda b, n: (b, n, 0)),
            pl.BlockSpec((1, ROWS, Fp), lambda b, n: (b, 0, 0)),
        ],
        out_specs=pl.BlockSpec((1, PB, 8), lambda b, n: (b, n, 0)),
        out_shape=jax.ShapeDtypeStruct((B, N, 8), jnp.float32),
        interpret=interpret,
    )(coords_pad, tri_pack)


def _sc_combine_body(canv_hbm, fidx_hbm, fid_hbm, wu_hbm, wv_hbm, ww_hbm,
                     ox_hbm, oy_hbm, oz_hbm,
                     canv_v, fidx_v, fid_v, wu_v, wv_v, ww_v, ox_v, oy_v, oz_v):
    wid = lax.axis_index("s") * NC + lax.axis_index("c")
    pltpu.sync_copy(canv_hbm, canv_v)
    pltpu.sync_copy(fidx_hbm, fidx_v)
    base = wid * PPW
    pltpu.sync_copy(fid_hbm.at[pl.ds(base, PPW)], fid_v)
    pltpu.sync_copy(wu_hbm.at[pl.ds(base, PPW)], wu_v)
    pltpu.sync_copy(wv_hbm.at[pl.ds(base, PPW)], wv_v)
    pltpu.sync_copy(ww_hbm.at[pl.ds(base, PPW)], ww_v)
    outs = (ox_v, oy_v, oz_v)
    for i in range(PPW // 16):
        f = fid_v[pl.ds(i * 16, 16)]
        u = wu_v[pl.ds(i * 16, 16)]
        v = wv_v[pl.ds(i * 16, 16)]
        w = ww_v[pl.ds(i * 16, 16)]
        ia = plsc.load_gather(fidx_v, [f * 3])
        ib = plsc.load_gather(fidx_v, [f * 3 + 1])
        ic = plsc.load_gather(fidx_v, [f * 3 + 2])
        for cdim in range(3):
            ca = plsc.load_gather(canv_v, [ia * 3 + cdim])
            cb = plsc.load_gather(canv_v, [ib * 3 + cdim])
            cc = plsc.load_gather(canv_v, [ic * 3 + cdim])
            outs[cdim][pl.ds(i * 16, 16)] = (u * ca + v * cb) + w * cc
    pltpu.sync_copy(ox_v, ox_hbm.at[pl.ds(base, PPW)])
    pltpu.sync_copy(oy_v, oy_hbm.at[pl.ds(base, PPW)])
    pltpu.sync_copy(oz_v, oz_hbm.at[pl.ds(base, PPW)])


def _sc_combine(canv_flat, fidx_flat, fid_flat, wu, wv, ww):
    kfn = pl.kernel(
        _sc_combine_body,
        out_type=[jax.ShapeDtypeStruct((B * N,), jnp.float32)] * 3,
        mesh=plsc.VectorSubcoreMesh(core_axis_name="c", subcore_axis_name="s"),
        compiler_params=_SC_PARAMS,
        scratch_types=[
            pltpu.VMEM((VFP,), jnp.float32),
            pltpu.VMEM((Fp * 3,), jnp.int32),
            pltpu.VMEM((PPW,), jnp.int32),
            pltpu.VMEM((PPW,), jnp.float32),
            pltpu.VMEM((PPW,), jnp.float32),
            pltpu.VMEM((PPW,), jnp.float32),
            pltpu.VMEM((PPW,), jnp.float32),
            pltpu.VMEM((PPW,), jnp.float32),
            pltpu.VMEM((PPW,), jnp.float32),
        ],
    )
    ox, oy, oz = kfn(canv_flat, fidx_flat, fid_flat, wu, wv, ww)
    return jnp.stack([ox, oy, oz], axis=-1)


@jax.jit
def kernel(coords, smpl_V, can_V, smpl_F):
    fidx_pad = jnp.concatenate(
        [smpl_F, jnp.broadcast_to(smpl_F[0:1], (Fp - F_, 3))], axis=0
    ).reshape(-1)
    vflat = jnp.pad(smpl_V.reshape(B, VF), ((0, 0), (0, VFP - VF))).reshape(-1)
    canv_flat = jnp.pad(can_V.reshape(VF), (0, VFP - VF))
    coords_pad = jnp.pad(coords, ((0, 0), (0, 0), (0, 5)))

    tri_pack = _sc_gather(vflat, fidx_pad)
    misc = _tc_query(coords_pad, tri_pack)

    sdf = misc[:, :, 0]
    wu = misc[:, :, 1].reshape(-1)
    wv = misc[:, :, 2].reshape(-1)
    ww = misc[:, :, 3].reshape(-1)
    fid_flat = misc[:, :, 4].astype(jnp.int32).reshape(-1)
    normal = misc[:, :, 5:8]

    out_coord = _sc_combine(canv_flat, fidx_pad, fid_flat, wu, wv, ww)
    out_coord = out_coord.reshape(B, N, 3)
    z = coords[..., 2:3]
    return (out_coord, sdf, normal, z)
